# baseline (speedup 1.0000x reference)
import jax
import jax.numpy as jnp
from jax.experimental import pallas as pl
from jax.experimental.pallas import tpu as pltpu

H = 1024
FF = 2816
E = 8
FF_TILE = 1408
N_FT = FF // FF_TILE
N_STEPS = E * N_FT
NSPLIT = 4
NBUF = 3


def _moe_kernel(x_ref, gate_ref, w1_hbm, w2_hbm, w3_hbm,
                out_ref, logits_ref,
                w1_buf, w2_buf, w3_buf, coef_ref, sems):
    HC = H // NSPLIT
    FC = FF_TILE // NSPLIT

    def issue(step, slot):
        e, f = step // N_FT, step % N_FT
        for c in range(NSPLIT):
            pltpu.make_async_copy(
                w1_hbm.at[e, pl.ds(c * HC, HC), pl.ds(f * FF_TILE, FF_TILE)],
                w1_buf.at[slot, pl.ds(c * HC, HC), :],
                sems.at[0, slot]).start()
            pltpu.make_async_copy(
                w2_hbm.at[e, pl.ds(f * FF_TILE + c * FC, FC), :],
                w2_buf.at[slot, pl.ds(c * FC, FC), :],
                sems.at[1, slot]).start()
            pltpu.make_async_copy(
                w3_hbm.at[e, pl.ds(c * HC, HC), pl.ds(f * FF_TILE, FF_TILE)],
                w3_buf.at[slot, pl.ds(c * HC, HC), :],
                sems.at[2, slot]).start()

    for s in range(NBUF - 1):
        issue(s, s)

    xf = x_ref[...]
    logits = jnp.dot(xf, gate_ref[...], preferred_element_type=jnp.float32)
    logits_ref[...] = logits
    idx = jax.lax.broadcasted_iota(jnp.int32, logits.shape, 1)
    v1 = jnp.max(logits, axis=1, keepdims=True)
    i1 = jnp.min(jnp.where(logits == v1, idx, E), axis=1, keepdims=True)
    oh1 = idx == i1
    masked = jnp.where(oh1, -jnp.inf, logits)
    v2 = jnp.max(masked, axis=1, keepdims=True)
    i2 = jnp.min(jnp.where(masked == v2, idx, E), axis=1, keepdims=True)
    oh2 = idx == i2
    p1 = 1.0 / (1.0 + jnp.exp(v2 - v1))
    coef = jnp.where(oh1, p1, 0.0) + jnp.where(oh2, 1.0 - p1, 0.0)

    x = xf.astype(jnp.bfloat16)
    acc = jnp.zeros_like(out_ref)

    for step in range(N_STEPS):
        slot = step % NBUF
        e = step // N_FT
        for c in range(NSPLIT):
            pltpu.make_async_copy(
                w1_hbm.at[0, pl.ds(0, HC), pl.ds(0, FF_TILE)],
                w1_buf.at[slot, pl.ds(0, HC), :], sems.at[0, slot]).wait()
            pltpu.make_async_copy(
                w2_hbm.at[0, pl.ds(0, FC), :],
                w2_buf.at[slot, pl.ds(0, FC), :], sems.at[1, slot]).wait()
            pltpu.make_async_copy(
                w3_hbm.at[0, pl.ds(0, HC), pl.ds(0, FF_TILE)],
                w3_buf.at[slot, pl.ds(0, HC), :], sems.at[2, slot]).wait()

        w1b = w1_buf[slot].astype(jnp.bfloat16)
        w3b = w3_buf[slot].astype(jnp.bfloat16)
        w2b = w2_buf[slot].astype(jnp.bfloat16)
        h = jax.nn.silu(jnp.dot(x, w1b, preferred_element_type=jnp.float32))
        h = h * jnp.dot(x, w3b, preferred_element_type=jnp.float32)
        contrib = jnp.dot(h.astype(jnp.bfloat16), w2b,
                          preferred_element_type=jnp.float32)
        ce = coef[:, e][:, None]
        acc = acc + ce * contrib

        if step + NBUF - 1 < N_STEPS:
            issue(step + NBUF - 1, (step + NBUF - 1) % NBUF)

    out_ref[...] = acc


@jax.jit
def kernel(hidden_states, gate_w, w1, w2, w3):
    B, S, _ = hidden_states.shape
    T = B * S
    x = hidden_states.reshape(T, H)

    out, logits = pl.pallas_call(
        _moe_kernel,
        in_specs=[
            pl.BlockSpec(memory_space=pltpu.VMEM),
            pl.BlockSpec(memory_space=pltpu.VMEM),
            pl.BlockSpec(memory_space=pl.ANY),
            pl.BlockSpec(memory_space=pl.ANY),
            pl.BlockSpec(memory_space=pl.ANY),
        ],
        out_specs=[
            pl.BlockSpec(memory_space=pltpu.VMEM),
            pl.BlockSpec(memory_space=pltpu.VMEM),
        ],
        out_shape=[
            jax.ShapeDtypeStruct((T, H), jnp.float32),
            jax.ShapeDtypeStruct((T, E), jnp.float32),
        ],
        scratch_shapes=[
            pltpu.VMEM((NBUF, H, FF_TILE), jnp.float32),
            pltpu.VMEM((NBUF, FF_TILE, H), jnp.float32),
            pltpu.VMEM((NBUF, H, FF_TILE), jnp.float32),
            pltpu.VMEM((T, E), jnp.float32),
            pltpu.SemaphoreType.DMA((3, NBUF)),
        ],
    )(x, gate_w, w1, w2, w3)

    return out.reshape(B, S, H), logits.reshape(B, S, E)


# speedup vs baseline: 1.0620x; 1.0620x over previous
import jax
import jax.numpy as jnp
from jax.experimental import pallas as pl
from jax.experimental.pallas import tpu as pltpu

H = 1024
FF = 2816
E = 8
CHUNK = 256
NCH = FF // CHUNK
N_CHUNKS = E * NCH
K = 8


def _moe_kernel(x_ref, gate_ref, w1_hbm, w2_hbm, w3_hbm,
                out_ref, logits_ref,
                w1_r, w2_r, w3_r, coef_ref, sems):
    def issue(ci, slot):
        e = ci // NCH
        c = ci - e * NCH
        pltpu.make_async_copy(
            w1_hbm.at[e, :, pl.ds(c * CHUNK, CHUNK)],
            w1_r.at[slot], sems.at[0, slot]).start()
        pltpu.make_async_copy(
            w2_hbm.at[e, pl.ds(c * CHUNK, CHUNK), :],
            w2_r.at[slot], sems.at[1, slot]).start()
        pltpu.make_async_copy(
            w3_hbm.at[e, :, pl.ds(c * CHUNK, CHUNK)],
            w3_r.at[slot], sems.at[2, slot]).start()

    for ci in range(K - 1):
        issue(ci, ci)

    xf = x_ref[...]
    logits = jnp.dot(xf, gate_ref[...], preferred_element_type=jnp.float32)
    logits_ref[...] = logits
    idx = jax.lax.broadcasted_iota(jnp.int32, logits.shape, 1)
    v1 = jnp.max(logits, axis=1, keepdims=True)
    i1 = jnp.min(jnp.where(logits == v1, idx, E), axis=1, keepdims=True)
    oh1 = idx == i1
    masked = jnp.where(oh1, -jnp.inf, logits)
    v2 = jnp.max(masked, axis=1, keepdims=True)
    i2 = jnp.min(jnp.where(masked == v2, idx, E), axis=1, keepdims=True)
    oh2 = idx == i2
    p1 = 1.0 / (1.0 + jnp.exp(v2 - v1))
    coef_ref[...] = jnp.where(oh1, p1, 0.0) + jnp.where(oh2, 1.0 - p1, 0.0)

    x = xf.astype(jnp.bfloat16)
    out_ref[...] = jnp.zeros_like(out_ref)
    coef = coef_ref[...]
    lane = jax.lax.broadcasted_iota(jnp.int32, coef.shape, 1)

    def body(ci, carry):
        slot = jax.lax.rem(ci, K)
        e = ci // NCH
        pltpu.make_async_copy(
            w1_hbm.at[0, :, pl.ds(0, CHUNK)],
            w1_r.at[slot], sems.at[0, slot]).wait()
        pltpu.make_async_copy(
            w2_hbm.at[0, pl.ds(0, CHUNK), :],
            w2_r.at[slot], sems.at[1, slot]).wait()
        pltpu.make_async_copy(
            w3_hbm.at[0, :, pl.ds(0, CHUNK)],
            w3_r.at[slot], sems.at[2, slot]).wait()

        w1b = w1_r[slot].astype(jnp.bfloat16)
        w3b = w3_r[slot].astype(jnp.bfloat16)
        w2b = w2_r[slot].astype(jnp.bfloat16)
        h = jax.nn.silu(jnp.dot(x, w1b, preferred_element_type=jnp.float32))
        h = h * jnp.dot(x, w3b, preferred_element_type=jnp.float32)
        ce = jnp.sum(jnp.where(lane == e, coef, 0.0), axis=1, keepdims=True)
        hb = (ce * h).astype(jnp.bfloat16)
        out_ref[...] += jnp.dot(hb, w2b, preferred_element_type=jnp.float32)

        nxt = ci + K - 1

        @pl.when(nxt < N_CHUNKS)
        def _():
            issue(nxt, jax.lax.rem(nxt, K))

        return carry

    jax.lax.fori_loop(0, N_CHUNKS, body, 0)


@jax.jit
def kernel(hidden_states, gate_w, w1, w2, w3):
    B, S, _ = hidden_states.shape
    T = B * S
    x = hidden_states.reshape(T, H)

    out, logits = pl.pallas_call(
        _moe_kernel,
        in_specs=[
            pl.BlockSpec(memory_space=pltpu.VMEM),
            pl.BlockSpec(memory_space=pltpu.VMEM),
            pl.BlockSpec(memory_space=pl.ANY),
            pl.BlockSpec(memory_space=pl.ANY),
            pl.BlockSpec(memory_space=pl.ANY),
        ],
        out_specs=[
            pl.BlockSpec(memory_space=pltpu.VMEM),
            pl.BlockSpec(memory_space=pltpu.VMEM),
        ],
        out_shape=[
            jax.ShapeDtypeStruct((T, H), jnp.float32),
            jax.ShapeDtypeStruct((T, E), jnp.float32),
        ],
        scratch_shapes=[
            pltpu.VMEM((K, H, CHUNK), jnp.float32),
            pltpu.VMEM((K, CHUNK, H), jnp.float32),
            pltpu.VMEM((K, H, CHUNK), jnp.float32),
            pltpu.VMEM((T, E), jnp.float32),
            pltpu.SemaphoreType.DMA((3, K)),
        ],
    )(x, gate_w, w1, w2, w3)

    return out.reshape(B, S, H), logits.reshape(B, S, E)
